# Initial kernel scaffold; baseline (speedup 1.0000x reference)
import jax
import jax.numpy as jnp
from jax.experimental import pallas as pl


def kernel(features, activated, W, b):
    raise NotImplementedError("write your pallas kernel here")


# speedup vs baseline: 1.0701x; 1.0701x over previous
import functools

import jax
import jax.numpy as jnp
from jax import lax
from jax.experimental import pallas as pl

TOP_K = 8
N_EXPERTS = 64
IN_FEATURES = 1024
OUT_FEATURES = 64
N_TOKENS = 2048

BLK_N = 128


def _body(f_ref, a_ref, w_ref, bflat_ref, out_ref):
    acc = jnp.dot(f_ref[...], w_ref[...], preferred_element_type=jnp.float32)
    acc = acc + bflat_ref[...]

    a = a_ref[...]
    lane = lax.broadcasted_iota(jnp.int32, (BLK_N, N_EXPERTS), 1)
    sel = []
    for _ in range(TOP_K):
        m = jnp.min(a, axis=1, keepdims=True)
        cand = jnp.where(a == m, lane, N_EXPERTS)
        amin = jnp.min(cand, axis=1, keepdims=True)
        sel.append(amin)
        a = jnp.where(lane == amin, jnp.inf, a)

    for k in range(TOP_K):
        e = sel[k]
        cur = acc
        width = (N_EXPERTS // 2) * OUT_FEATURES
        for bit in range(5, -1, -1):
            take_hi = ((e >> bit) & 1) == 1
            cur = jnp.where(take_hi, cur[:, width:], cur[:, :width])
            width //= 2
        out_ref[:, k * OUT_FEATURES:(k + 1) * OUT_FEATURES] = cur


@jax.jit
def kernel(features, activated, W, b):
    wr = W.transpose(1, 0, 2).reshape(IN_FEATURES, N_EXPERTS * OUT_FEATURES)
    wr = wr.astype(jnp.bfloat16)
    f = features.astype(jnp.bfloat16)
    bflat = b.reshape(1, N_EXPERTS * OUT_FEATURES)

    out = pl.pallas_call(
        _body,
        grid=(N_TOKENS // BLK_N,),
        in_specs=[
            pl.BlockSpec((BLK_N, IN_FEATURES), lambda i: (i, 0)),
            pl.BlockSpec((BLK_N, N_EXPERTS), lambda i: (i, 0)),
            pl.BlockSpec((IN_FEATURES, N_EXPERTS * OUT_FEATURES),
                         lambda i: (0, 0)),
            pl.BlockSpec((1, N_EXPERTS * OUT_FEATURES), lambda i: (0, 0)),
        ],
        out_specs=pl.BlockSpec((BLK_N, TOP_K * OUT_FEATURES), lambda i: (i, 0)),
        out_shape=jax.ShapeDtypeStruct(
            (N_TOKENS, TOP_K * OUT_FEATURES), jnp.float32),
    )(f, activated, wr, bflat)
    return out.reshape(N_TOKENS, TOP_K, OUT_FEATURES)
